# baseline (speedup 1.0000x reference)
import jax
import jax.numpy as jnp
from jax.experimental import pallas as pl

N = 8192
D = 1024
BN = 512


def _fused_kernel(xs_ref, wd_ref, bd_ref, wl_ref, bl_ref, wr_ref, br_ref,
                  out_ref):
    x = xs_ref[...]
    dec = jnp.dot(x, wd_ref[...],
                  preferred_element_type=jnp.float32) + bd_ref[0, 0]
    xb = x.astype(jnp.bfloat16)
    l = jnp.dot(xb, wl_ref[...], preferred_element_type=jnp.float32) + bl_ref[...]
    r = jnp.dot(xb, wr_ref[...], preferred_element_type=jnp.float32) + br_ref[...]
    out_ref[...] = jnp.where(dec > 0.0, r, l)


def kernel(xs, w_dec, b_dec, W_left, b_left, W_right, b_right):
    wd = w_dec.reshape(D, 1)
    bd = b_dec.reshape(1, 1)
    bl = b_left.reshape(1, D)
    br = b_right.reshape(1, D)
    wl16 = W_left.astype(jnp.bfloat16)
    wr16 = W_right.astype(jnp.bfloat16)
    grid = (N // BN,)
    return pl.pallas_call(
        _fused_kernel,
        grid=grid,
        in_specs=[
            pl.BlockSpec((BN, D), lambda i: (i, 0)),
            pl.BlockSpec((D, 1), lambda i: (0, 0)),
            pl.BlockSpec((1, 1), lambda i: (0, 0)),
            pl.BlockSpec((D, D), lambda i: (0, 0)),
            pl.BlockSpec((1, D), lambda i: (0, 0)),
            pl.BlockSpec((D, D), lambda i: (0, 0)),
            pl.BlockSpec((1, D), lambda i: (0, 0)),
        ],
        out_specs=pl.BlockSpec((BN, D), lambda i: (i, 0)),
        out_shape=jax.ShapeDtypeStruct((N, D), jnp.float32),
    )(xs, wd, bd, wl16, bl, wr16, br)


# speedup vs baseline: 1.1087x; 1.1087x over previous
import jax
import jax.numpy as jnp
from jax.experimental import pallas as pl

N = 8192
D = 1024
BN = 1024


def _fused_kernel(xs_ref, wd_ref, bd_ref, wl_ref, bl_ref, wr_ref, br_ref,
                  out_ref):
    x = xs_ref[...]
    dec = jnp.dot(x, wd_ref[...],
                  preferred_element_type=jnp.float32) + bd_ref[0, 0]
    xb = x.astype(jnp.bfloat16)
    l = jnp.dot(xb, wl_ref[...].astype(jnp.bfloat16),
                preferred_element_type=jnp.float32) + bl_ref[...]
    r = jnp.dot(xb, wr_ref[...].astype(jnp.bfloat16),
                preferred_element_type=jnp.float32) + br_ref[...]
    out_ref[...] = jnp.where(dec > 0.0, r, l)


def kernel(xs, w_dec, b_dec, W_left, b_left, W_right, b_right):
    wd = w_dec.reshape(D, 1)
    bd = b_dec.reshape(1, 1)
    bl = b_left.reshape(1, D)
    br = b_right.reshape(1, D)
    wl16 = W_left
    wr16 = W_right
    grid = (N // BN,)
    return pl.pallas_call(
        _fused_kernel,
        grid=grid,
        in_specs=[
            pl.BlockSpec((BN, D), lambda i: (i, 0)),
            pl.BlockSpec((D, 1), lambda i: (0, 0)),
            pl.BlockSpec((1, 1), lambda i: (0, 0)),
            pl.BlockSpec((D, D), lambda i: (0, 0)),
            pl.BlockSpec((1, D), lambda i: (0, 0)),
            pl.BlockSpec((D, D), lambda i: (0, 0)),
            pl.BlockSpec((1, D), lambda i: (0, 0)),
        ],
        out_specs=pl.BlockSpec((BN, D), lambda i: (i, 0)),
        out_shape=jax.ShapeDtypeStruct((N, D), jnp.float32),
    )(xs, wd, bd, wl16, bl, wr16, br)
